# Initial kernel scaffold; baseline (speedup 1.0000x reference)
import jax
import jax.numpy as jnp
from jax.experimental import pallas as pl


def kernel(rules, global_to_local, weights):
    raise NotImplementedError("write your pallas kernel here")


# speedup vs baseline: 676.5374x; 676.5374x over previous
import functools

import jax
import jax.numpy as jnp
from jax import lax
from jax.experimental import pallas as pl
from jax.experimental.pallas import tpu as pltpu
from jax.experimental.pallas import tpu_sc as plsc

NC = 2
NS = 16
LANES = 16


def _noisy_or(rules_flat, g2l_pad, w_pad, *, B, L, T_pad, W_pad, num_rel):
    NW = NC * NS
    chunk = T_pad // NS
    rows_w = B // NW
    groups = rows_w // LANES
    gl = LANES * L

    mesh = plsc.VectorSubcoreMesh(core_axis_name="c", subcore_axis_name="s")

    @functools.partial(
        pl.kernel,
        out_type=jax.ShapeDtypeStruct((B,), jnp.float32),
        mesh=mesh,
        compiler_params=pltpu.CompilerParams(needs_layout_passes=False),
        scratch_types=[
            pltpu.VMEM((W_pad,), jnp.float32),
            pltpu.VMEM((chunk,), jnp.int32),
            pltpu.VMEM((T_pad,), jnp.float32),
            pltpu.VMEM_SHARED((T_pad,), jnp.float32),
            pltpu.VMEM((gl,), jnp.int32),
            pltpu.VMEM((gl,), jnp.int32),
            pltpu.VMEM((rows_w,), jnp.float32),
            pltpu.SemaphoreType.DMA,
            pltpu.SemaphoreType.DMA,
        ],
    )
    def run(rules_hbm, g2l_hbm, w_hbm, out_hbm,
            w_v, g2l_v, t_v, t_sh, r0_v, r1_v, o_v, sem0, sem1):
        cid = lax.axis_index("c")
        sid = lax.axis_index("s")
        wid = sid * NC + cid

        pltpu.sync_copy(w_hbm, w_v)
        pltpu.sync_copy(g2l_hbm.at[pl.ds(sid * chunk, chunk)], g2l_v)

        def build(i, carry):
            idx = g2l_v[pl.ds(i * LANES, LANES)]
            w = plsc.load_gather(w_v, [idx])
            f = 1.0 / (1.0 + jnp.exp(w))
            f = jnp.where(idx == num_rel, 1.0, f)
            t_v[pl.ds(i * LANES, LANES)] = f
            return carry

        lax.fori_loop(0, chunk // LANES, build, 0)
        pltpu.sync_copy(t_v.at[pl.ds(0, chunk)],
                        t_sh.at[pl.ds(sid * chunk, chunk)])
        plsc.subcore_barrier()
        pltpu.sync_copy(t_sh, t_v)

        row_base = wid * rows_w
        biota = lax.iota(jnp.int32, LANES) * L

        sems = [sem0, sem1]
        bufs = [r0_v, r1_v]
        handles = [None, None]

        def start(g):
            b = g & 1
            src = rules_hbm.at[pl.ds((row_base + g * LANES) * L, gl)]
            handles[b] = pltpu.async_copy(src, bufs[b], sems[b])

        start(0)
        for g in range(groups):
            b = g & 1
            if g + 1 < groups:
                start(g + 1)
            handles[b].wait()
            rbuf = bufs[b]

            def step(i, accs):
                a0, a1, a2, a3 = accs
                l = i * 4
                i0 = plsc.load_gather(rbuf, [biota + l])
                i1 = plsc.load_gather(rbuf, [biota + (l + 1)])
                i2 = plsc.load_gather(rbuf, [biota + (l + 2)])
                i3 = plsc.load_gather(rbuf, [biota + (l + 3)])
                f0 = plsc.load_gather(t_v, [i0])
                f1 = plsc.load_gather(t_v, [i1])
                f2 = plsc.load_gather(t_v, [i2])
                f3 = plsc.load_gather(t_v, [i3])
                return (a0 * f0, a1 * f1, a2 * f2, a3 * f3)

            ones = jnp.ones((LANES,), jnp.float32)
            a0, a1, a2, a3 = lax.fori_loop(0, L // 4, step,
                                           (ones, ones, ones, ones))
            prod = (a0 * a1) * (a2 * a3)
            res = jnp.clip(1.0 - prod, 0.0001, 0.99999)
            o_v[pl.ds(g * LANES, LANES)] = res

        pltpu.sync_copy(o_v, out_hbm.at[pl.ds(row_base, rows_w)])

    return run(rules_flat, g2l_pad, w_pad)


def kernel(rules, global_to_local, weights):
    B, L = rules.shape
    num_rel = weights.shape[0] - 1
    n_g2l = global_to_local.shape[0]

    chunk_unit = NS * LANES
    T_pad = ((n_g2l + chunk_unit - 1) // chunk_unit) * chunk_unit
    W_pad = ((num_rel + 1 + 127) // 128) * 128

    rules_flat = rules.reshape(-1)
    g2l_pad = jnp.concatenate(
        [global_to_local,
         jnp.full((T_pad - n_g2l,), num_rel, dtype=global_to_local.dtype)])
    w_pad = jnp.concatenate(
        [weights[:, 0], jnp.zeros((W_pad - num_rel - 1,), weights.dtype)])

    out = _noisy_or(rules_flat, g2l_pad, w_pad,
                    B=B, L=L, T_pad=T_pad, W_pad=W_pad, num_rel=num_rel)
    return out.reshape(B, 1)


# speedup vs baseline: 743.8508x; 1.0995x over previous
import functools

import jax
import jax.numpy as jnp
from jax import lax
from jax.experimental import pallas as pl
from jax.experimental.pallas import tpu as pltpu
from jax.experimental.pallas import tpu_sc as plsc

NC = 2
NS = 16
LANES = 16


def _noisy_or(rules_flat, g2l, w_flat, *, B, L, num_rel):
    NW = NC * NS
    n_g2l = g2l.shape[0]
    n_ids = n_g2l - 1
    chunk_unit = NS * LANES
    T_pad = ((n_ids + chunk_unit - 1) // chunk_unit) * chunk_unit
    chunk = T_pad // NS
    tail = n_ids - (NS - 1) * chunk
    assert 0 < tail <= chunk and tail % 8 == 0
    w_copy = (num_rel + 1) // 8 * 8
    W_pad = ((num_rel + 1 + 127) // 128) * 128
    rows_w = B // NW
    groups = rows_w // LANES
    gl = LANES * L

    mesh = plsc.VectorSubcoreMesh(core_axis_name="c", subcore_axis_name="s")

    @functools.partial(
        pl.kernel,
        out_type=jax.ShapeDtypeStruct((B,), jnp.float32),
        mesh=mesh,
        compiler_params=pltpu.CompilerParams(needs_layout_passes=False),
        scratch_types=[
            pltpu.VMEM((W_pad,), jnp.float32),
            pltpu.VMEM((chunk,), jnp.int32),
            pltpu.VMEM((T_pad,), jnp.float32),
            pltpu.VMEM_SHARED((T_pad,), jnp.float32),
            pltpu.VMEM((gl,), jnp.int32),
            pltpu.VMEM((gl,), jnp.int32),
            pltpu.VMEM((rows_w,), jnp.float32),
            pltpu.SemaphoreType.DMA,
            pltpu.SemaphoreType.DMA,
        ],
    )
    def run(rules_hbm, g2l_hbm, w_hbm, out_hbm,
            w_v, g2l_v, t_v, t_sh, r0_v, r1_v, o_v, sem0, sem1):
        cid = lax.axis_index("c")
        sid = lax.axis_index("s")
        wid = sid * NC + cid

        with jax.named_scope("build"):
            pltpu.sync_copy(w_hbm.at[pl.ds(0, w_copy)],
                            w_v.at[pl.ds(0, w_copy)])

            @pl.when(sid < NS - 1)
            def _():
                pltpu.sync_copy(g2l_hbm.at[pl.ds(sid * chunk, chunk)], g2l_v)

            @pl.when(sid == NS - 1)
            def _():
                pltpu.sync_copy(g2l_hbm.at[pl.ds((NS - 1) * chunk, tail)],
                                g2l_v.at[pl.ds(0, tail)])

            limit = jnp.where(sid == NS - 1, tail, chunk)
            lane = lax.iota(jnp.int32, LANES)

            @plsc.parallel_loop(0, chunk // LANES, unroll=2)
            def _(i):
                idx = g2l_v[pl.ds(i * LANES, LANES)]
                idx = jnp.where(i * LANES + lane < limit, idx, num_rel)
                w = plsc.load_gather(w_v, [idx])
                f = 1.0 / (1.0 + jnp.exp(w))
                f = jnp.where(idx == num_rel, 1.0, f)
                t_v[pl.ds(i * LANES, LANES)] = f

        with jax.named_scope("bcast"):
            pltpu.sync_copy(t_v.at[pl.ds(0, chunk)],
                            t_sh.at[pl.ds(sid * chunk, chunk)])
            plsc.subcore_barrier()
            pltpu.sync_copy(t_sh, t_v)

        row_base = wid * rows_w
        biota = lax.iota(jnp.int32, LANES) * L

        sems = [sem0, sem1]
        bufs = [r0_v, r1_v]
        handles = [None, None]

        def start(g):
            b = g & 1
            src = rules_hbm.at[pl.ds((row_base + g * LANES) * L, gl)]
            handles[b] = pltpu.async_copy(src, bufs[b], sems[b])

        with jax.named_scope("main"):
            start(0)
            for g in range(groups):
                b = g & 1
                if g + 1 < groups:
                    start(g + 1)
                handles[b].wait()
                rbuf = bufs[b]

                ones = jnp.ones((LANES,), jnp.float32)

                @plsc.parallel_loop(0, L // 4, unroll=2,
                                    carry=(ones, ones, ones, ones))
                def accs(i, carry):
                    a0, a1, a2, a3 = carry
                    l = i * 4
                    i0 = plsc.load_gather(rbuf, [biota + l])
                    i1 = plsc.load_gather(rbuf, [biota + (l + 1)])
                    i2 = plsc.load_gather(rbuf, [biota + (l + 2)])
                    i3 = plsc.load_gather(rbuf, [biota + (l + 3)])
                    f0 = plsc.load_gather(t_v, [i0])
                    f1 = plsc.load_gather(t_v, [i1])
                    f2 = plsc.load_gather(t_v, [i2])
                    f3 = plsc.load_gather(t_v, [i3])
                    return (a0 * f0, a1 * f1, a2 * f2, a3 * f3)

                a0, a1, a2, a3 = accs
                prod = (a0 * a1) * (a2 * a3)
                res = jnp.clip(1.0 - prod, 0.0001, 0.99999)
                o_v[pl.ds(g * LANES, LANES)] = res

            pltpu.sync_copy(o_v, out_hbm.at[pl.ds(row_base, rows_w)])

    return run(rules_flat, g2l, w_flat)


def kernel(rules, global_to_local, weights):
    B, L = rules.shape
    num_rel = weights.shape[0] - 1
    out = _noisy_or(rules.reshape(-1), global_to_local,
                    weights.reshape(-1), B=B, L=L, num_rel=num_rel)
    return out.reshape(B, 1)


# speedup vs baseline: 759.6840x; 1.0213x over previous
import functools

import jax
import jax.numpy as jnp
from jax import lax
from jax.experimental import pallas as pl
from jax.experimental.pallas import tpu as pltpu
from jax.experimental.pallas import tpu_sc as plsc

NC = 2
NS = 16
LANES = 16
NBUF = 2


def _noisy_or(rules_flat, g2l, w_flat, *, B, L, num_rel):
    NW = NC * NS
    n_g2l = g2l.shape[0]
    n_ids = n_g2l - 1
    chunk_unit = NS * LANES
    T_pad = ((n_ids + chunk_unit - 1) // chunk_unit) * chunk_unit
    chunk = T_pad // NS
    tail = n_ids - (NS - 1) * chunk
    assert 0 < tail <= chunk and tail % 8 == 0
    w_copy = (num_rel + 1) // 8 * 8
    W_pad = ((num_rel + 1 + chunk_unit - 1) // chunk_unit) * chunk_unit
    f_chunk = W_pad // NS
    rows_w = B // NW
    groups = rows_w // LANES
    assert groups % NBUF == 0
    gl = LANES * L

    mesh = plsc.VectorSubcoreMesh(core_axis_name="c", subcore_axis_name="s")

    @functools.partial(
        pl.kernel,
        out_type=jax.ShapeDtypeStruct((B,), jnp.float32),
        mesh=mesh,
        compiler_params=pltpu.CompilerParams(needs_layout_passes=False),
        scratch_types=[
            pltpu.VMEM((W_pad,), jnp.float32),
            pltpu.VMEM((chunk,), jnp.int32),
            pltpu.VMEM((T_pad,), jnp.float32),
            pltpu.VMEM_SHARED((T_pad,), jnp.float32),
            [pltpu.VMEM((gl,), jnp.int32) for _ in range(NBUF)],
            pltpu.VMEM((rows_w,), jnp.float32),
            [pltpu.SemaphoreType.DMA for _ in range(NBUF)],
        ],
    )
    def run(rules_hbm, g2l_hbm, w_hbm, out_hbm,
            w_v, g2l_v, t_v, t_sh, rbufs, o_v, sems):
        cid = lax.axis_index("c")
        sid = lax.axis_index("s")
        wid = sid * NC + cid
        lane = lax.iota(jnp.int32, LANES)
        row_base = wid * rows_w

        def rules_src(g):
            return rules_hbm.at[pl.ds((row_base + g * LANES) * L, gl)]

        for b in range(NBUF):
            pltpu.async_copy(rules_src(b), rbufs[b], sems[b])

        with jax.named_scope("build"):
            pltpu.sync_copy(w_hbm.at[pl.ds(0, w_copy)],
                            w_v.at[pl.ds(0, w_copy)])

            @pl.when(sid < NS - 1)
            def _():
                pltpu.sync_copy(g2l_hbm.at[pl.ds(sid * chunk, chunk)], g2l_v)

            @pl.when(sid == NS - 1)
            def _():
                pltpu.sync_copy(g2l_hbm.at[pl.ds((NS - 1) * chunk, tail)],
                                g2l_v.at[pl.ds(0, tail)])

            f_base = sid * f_chunk

            @plsc.parallel_loop(0, f_chunk // LANES, unroll=2)
            def _(i):
                w = w_v[pl.ds(f_base + i * LANES, LANES)]
                f = 1.0 / (1.0 + jnp.exp(w))
                f = jnp.where(f_base + i * LANES + lane >= num_rel, 1.0, f)
                w_v[pl.ds(f_base + i * LANES, LANES)] = f

            pltpu.sync_copy(w_v.at[pl.ds(f_base, f_chunk)],
                            t_sh.at[pl.ds(f_base, f_chunk)])
            plsc.subcore_barrier()
            pltpu.sync_copy(t_sh.at[pl.ds(0, W_pad)], w_v)
            plsc.subcore_barrier()

            limit = jnp.where(sid == NS - 1, tail, chunk)

            @plsc.parallel_loop(0, chunk // LANES, unroll=2)
            def _(i):
                idx = g2l_v[pl.ds(i * LANES, LANES)]
                idx = jnp.where(i * LANES + lane < limit, idx, num_rel)
                t_v[pl.ds(i * LANES, LANES)] = plsc.load_gather(w_v, [idx])

        with jax.named_scope("bcast"):
            pltpu.sync_copy(t_v.at[pl.ds(0, chunk)],
                            t_sh.at[pl.ds(sid * chunk, chunk)])
            plsc.subcore_barrier()
            pltpu.sync_copy(t_sh, t_v)

        biota = lane * L
        ones = jnp.ones((LANES,), jnp.float32)

        def body(j, carry):
            for sub in range(NBUF):
                g = j * NBUF + sub
                rbuf, sem = rbufs[sub], sems[sub]
                pltpu.make_async_copy(rules_src(0), rbuf, sem).wait()

                @plsc.parallel_loop(0, L // 4, unroll=2,
                                    carry=(ones, ones, ones, ones))
                def accs(i, c):
                    a0, a1, a2, a3 = c
                    l = i * 4
                    i0 = plsc.load_gather(rbuf, [biota + l])
                    i1 = plsc.load_gather(rbuf, [biota + (l + 1)])
                    i2 = plsc.load_gather(rbuf, [biota + (l + 2)])
                    i3 = plsc.load_gather(rbuf, [biota + (l + 3)])
                    f0 = plsc.load_gather(t_v, [i0])
                    f1 = plsc.load_gather(t_v, [i1])
                    f2 = plsc.load_gather(t_v, [i2])
                    f3 = plsc.load_gather(t_v, [i3])
                    return (a0 * f0, a1 * f1, a2 * f2, a3 * f3)

                @pl.when(g + NBUF < groups)
                def _():
                    pltpu.async_copy(rules_src(g + NBUF), rbuf, sem)

                a0, a1, a2, a3 = accs
                prod = (a0 * a1) * (a2 * a3)
                res = jnp.clip(1.0 - prod, 0.0001, 0.99999)
                o_v[pl.ds(g * LANES, LANES)] = res
            return carry

        with jax.named_scope("main"):
            lax.fori_loop(0, groups // NBUF, body, 0)
            pltpu.sync_copy(o_v, out_hbm.at[pl.ds(row_base, rows_w)])

    return run(rules_flat, g2l, w_flat)


def kernel(rules, global_to_local, weights):
    B, L = rules.shape
    num_rel = weights.shape[0] - 1
    out = _noisy_or(rules.reshape(-1), global_to_local,
                    weights.reshape(-1), B=B, L=L, num_rel=num_rel)
    return out.reshape(B, 1)
